# baseline (speedup 1.0000x reference)
import jax
import jax.numpy as jnp
from jax.experimental import pallas as pl
from jax.experimental.pallas import tpu as pltpu

N = 4096
BM = 1024
BMM = 1024
KB = 1024
NK = N // KB


def _dsum_kernel(a_ref, d_ref, abf_ref):
    a = a_ref[...]
    rs = jnp.sum(a, axis=1, keepdims=True)
    d_ref[...] = jnp.where(rs > 0, 1.0 / jnp.sqrt(rs), 0.0)
    abf_ref[...] = a.astype(jnp.bfloat16)


def _s1_kernel(x_ref, w_ref, d_ref, o_ref):
    s = jnp.dot(x_ref[...], w_ref[...], preferred_element_type=jnp.float32)
    o_ref[...] = (s * d_ref[...]).astype(jnp.bfloat16)


def _mid_kernel(a_ref, s1_ref, wcat_ref, d_ref, o_ref, acc_ref):
    k = pl.program_id(1)
    s1_blk = s1_ref[pl.ds(k * KB, KB), :]
    part = jnp.dot(a_ref[...], s1_blk, preferred_element_type=jnp.float32)

    @pl.when(k == 0)
    def _():
        acc_ref[...] = part

    @pl.when(k > 0)
    def _():
        acc_ref[...] += part

    @pl.when(k == NK - 1)
    def _():
        h = jnp.maximum(acc_ref[...] * d_ref[...], 0.0)
        s2 = jnp.dot(h, wcat_ref[...], preferred_element_type=jnp.float32)
        o_ref[...] = (s2 * d_ref[...]).astype(jnp.bfloat16)


def _out_kernel(a_ref, s2_ref, d_ref, m_ref, s_ref, acc_ref):
    k = pl.program_id(1)
    s2_blk = s2_ref[pl.ds(k * KB, KB), :]
    part = jnp.dot(a_ref[...], s2_blk, preferred_element_type=jnp.float32)

    @pl.when(k == 0)
    def _():
        acc_ref[...] = part

    @pl.when(k > 0)
    def _():
        acc_ref[...] += part

    @pl.when(k == NK - 1)
    def _():
        out = acc_ref[...] * d_ref[...]
        m_ref[...] = out[:, :128]
        s_ref[...] = out[:, 128:]


def kernel(adj, input, W1, Wm, Ws):
    x = jnp.squeeze(input)
    f_in = x.shape[1]
    h_dim = W1.shape[1]
    out_dim = Wm.shape[1]
    wcat = jnp.concatenate([Wm, Ws], axis=1)

    d, a_bf = pl.pallas_call(
        _dsum_kernel,
        grid=(N // BM,),
        in_specs=[pl.BlockSpec((BM, N), lambda i: (i, 0))],
        out_specs=[
            pl.BlockSpec((BM, 1), lambda i: (i, 0)),
            pl.BlockSpec((BM, N), lambda i: (i, 0)),
        ],
        out_shape=[
            jax.ShapeDtypeStruct((N, 1), jnp.float32),
            jax.ShapeDtypeStruct((N, N), jnp.bfloat16),
        ],
        compiler_params=pltpu.CompilerParams(dimension_semantics=("parallel",)),
    )(adj)

    s1 = pl.pallas_call(
        _s1_kernel,
        in_specs=[
            pl.BlockSpec((N, f_in), lambda: (0, 0)),
            pl.BlockSpec((f_in, h_dim), lambda: (0, 0)),
            pl.BlockSpec((N, 1), lambda: (0, 0)),
        ],
        out_specs=pl.BlockSpec((N, h_dim), lambda: (0, 0)),
        out_shape=jax.ShapeDtypeStruct((N, h_dim), jnp.bfloat16),
    )(x, W1, d)

    s2 = pl.pallas_call(
        _mid_kernel,
        grid=(N // BMM, NK),
        in_specs=[
            pl.BlockSpec((BMM, KB), lambda i, k: (i, k)),
            pl.BlockSpec((N, h_dim), lambda i, k: (0, 0)),
            pl.BlockSpec((h_dim, 2 * out_dim), lambda i, k: (0, 0)),
            pl.BlockSpec((BMM, 1), lambda i, k: (i, 0)),
        ],
        out_specs=pl.BlockSpec((BMM, 2 * out_dim), lambda i, k: (i, 0)),
        out_shape=jax.ShapeDtypeStruct((N, 2 * out_dim), jnp.bfloat16),
        scratch_shapes=[pltpu.VMEM((BMM, 2 * out_dim), jnp.float32)],
        compiler_params=pltpu.CompilerParams(
            dimension_semantics=("parallel", "arbitrary")
        ),
    )(a_bf, s1, wcat, d)

    z_mean, z_log = pl.pallas_call(
        _out_kernel,
        grid=(N // BMM, NK),
        in_specs=[
            pl.BlockSpec((BMM, KB), lambda i, k: (i, k)),
            pl.BlockSpec((N, 2 * out_dim), lambda i, k: (0, 0)),
            pl.BlockSpec((BMM, 1), lambda i, k: (i, 0)),
        ],
        out_specs=[
            pl.BlockSpec((BMM, out_dim), lambda i, k: (i, 0)),
            pl.BlockSpec((BMM, out_dim), lambda i, k: (i, 0)),
        ],
        out_shape=[
            jax.ShapeDtypeStruct((N, out_dim), jnp.float32),
            jax.ShapeDtypeStruct((N, out_dim), jnp.float32),
        ],
        scratch_shapes=[pltpu.VMEM((BMM, 2 * out_dim), jnp.float32)],
        compiler_params=pltpu.CompilerParams(
            dimension_semantics=("parallel", "arbitrary")
        ),
    )(a_bf, s2, d)

    return (z_mean, z_log)


# speedup vs baseline: 1.7522x; 1.7522x over previous
import jax
import jax.numpy as jnp
from jax.experimental import pallas as pl
from jax.experimental.pallas import tpu as pltpu

N = 4096
CB = 256
MB = 512
NCAST = N // CB
NMM = N // MB


def _gcn_kernel(a_ref, x_ref, w1_ref, wcat_ref, m_ref, s_ref,
                abf_ref, d_ref, s1_ref, s2_ref):
    t = pl.program_id(0)

    @pl.when(t < NCAST)
    def _cast_phase():
        a = a_ref[...]
        rs = jnp.sum(a, axis=1, keepdims=True)
        rows = pl.ds(t * CB, CB)
        d_ref[rows, :] = jnp.where(rs > 0, 1.0 / jnp.sqrt(rs), 0.0)
        abf_ref[rows, :] = a.astype(jnp.bfloat16)

    @pl.when(t == NCAST - 1)
    def _s1_phase():
        xw = jnp.dot(x_ref[...], w1_ref[...], preferred_element_type=jnp.float32)
        s1_ref[...] = (xw * d_ref[...]).astype(jnp.bfloat16)

    @pl.when((t >= NCAST) & (t < NCAST + NMM))
    def _mid_phase():
        rows = pl.ds((t - NCAST) * MB, MB)
        acc = jnp.dot(abf_ref[rows, :], s1_ref[...],
                      preferred_element_type=jnp.float32)
        dj = d_ref[rows, :]
        h = jnp.maximum(acc * dj, 0.0)
        s2 = jnp.dot(h, wcat_ref[...], preferred_element_type=jnp.float32)
        s2_ref[rows, :] = (s2 * dj).astype(jnp.bfloat16)

    @pl.when(t >= NCAST + NMM)
    def _out_phase():
        rows = pl.ds((t - NCAST - NMM) * MB, MB)
        acc = jnp.dot(abf_ref[rows, :], s2_ref[...],
                      preferred_element_type=jnp.float32)
        out = acc * d_ref[rows, :]
        m_ref[...] = out[:, :128]
        s_ref[...] = out[:, 128:]


def kernel(adj, input, W1, Wm, Ws):
    x = jnp.squeeze(input)
    f_in = x.shape[1]
    h_dim = W1.shape[1]
    out_dim = Wm.shape[1]
    wcat = jnp.concatenate([Wm, Ws], axis=1)

    z_mean, z_log = pl.pallas_call(
        _gcn_kernel,
        grid=(NCAST + 2 * NMM,),
        in_specs=[
            pl.BlockSpec((CB, N), lambda t: (jnp.minimum(t, NCAST - 1), 0)),
            pl.BlockSpec((N, f_in), lambda t: (0, 0)),
            pl.BlockSpec((f_in, h_dim), lambda t: (0, 0)),
            pl.BlockSpec((h_dim, 2 * out_dim), lambda t: (0, 0)),
        ],
        out_specs=[
            pl.BlockSpec(
                (MB, out_dim),
                lambda t: (jnp.clip(t - NCAST - NMM, 0, NMM - 1), 0),
            ),
            pl.BlockSpec(
                (MB, out_dim),
                lambda t: (jnp.clip(t - NCAST - NMM, 0, NMM - 1), 0),
            ),
        ],
        out_shape=[
            jax.ShapeDtypeStruct((N, out_dim), jnp.float32),
            jax.ShapeDtypeStruct((N, out_dim), jnp.float32),
        ],
        scratch_shapes=[
            pltpu.VMEM((N, N), jnp.bfloat16),
            pltpu.VMEM((N, 1), jnp.float32),
            pltpu.VMEM((N, h_dim), jnp.bfloat16),
            pltpu.VMEM((N, 2 * out_dim), jnp.bfloat16),
        ],
        compiler_params=pltpu.CompilerParams(
            dimension_semantics=("arbitrary",)
        ),
    )(adj, x, W1, wcat)

    return (z_mean, z_log)


# speedup vs baseline: 1.8172x; 1.0371x over previous
import jax
import jax.numpy as jnp
from jax.experimental import pallas as pl
from jax.experimental.pallas import tpu as pltpu

N = 4096
CB = 256
MB = 1024
NCAST = N // CB
NMM = N // MB


def _gcn_kernel(a_ref, x_ref, w1_ref, wcat_ref, m_ref, s_ref,
                abf_ref, d_ref, s1_ref, s2_ref, xw_ref):
    t = pl.program_id(0)

    @pl.when(t == 0)
    def _xw_phase():
        xw_ref[...] = jnp.dot(x_ref[...], w1_ref[...],
                              preferred_element_type=jnp.float32)

    @pl.when(t < NCAST)
    def _cast_phase():
        a = a_ref[...]
        rs = jnp.sum(a, axis=1, keepdims=True)
        rows = pl.ds(t * CB, CB)
        d_ref[rows, :] = jnp.where(rs > 0, 1.0 / jnp.sqrt(rs), 0.0)
        abf_ref[rows, :] = a.astype(jnp.bfloat16)

    @pl.when(t == NCAST - 1)
    def _s1_phase():
        s1_ref[...] = (xw_ref[...] * d_ref[...]).astype(jnp.bfloat16)

    @pl.when((t >= NCAST) & (t < NCAST + NMM))
    def _mid_phase():
        rows = pl.ds((t - NCAST) * MB, MB)
        acc = jnp.dot(abf_ref[rows, :], s1_ref[...],
                      preferred_element_type=jnp.float32)
        dj = d_ref[rows, :]
        h = jnp.maximum(acc * dj, 0.0)
        s2 = jnp.dot(h, wcat_ref[...], preferred_element_type=jnp.float32)
        s2_ref[rows, :] = (s2 * dj).astype(jnp.bfloat16)

    @pl.when(t >= NCAST + NMM)
    def _out_phase():
        rows = pl.ds((t - NCAST - NMM) * MB, MB)
        acc = jnp.dot(abf_ref[rows, :], s2_ref[...],
                      preferred_element_type=jnp.float32)
        out = acc * d_ref[rows, :]
        m_ref[...] = out[:, :128]
        s_ref[...] = out[:, 128:]


def kernel(adj, input, W1, Wm, Ws):
    x = jnp.squeeze(input)
    f_in = x.shape[1]
    h_dim = W1.shape[1]
    out_dim = Wm.shape[1]
    wcat = jnp.concatenate([Wm, Ws], axis=1)

    z_mean, z_log = pl.pallas_call(
        _gcn_kernel,
        grid=(NCAST + 2 * NMM,),
        in_specs=[
            pl.BlockSpec((CB, N), lambda t: (jnp.minimum(t, NCAST - 1), 0)),
            pl.BlockSpec((N, f_in), lambda t: (0, 0)),
            pl.BlockSpec((f_in, h_dim), lambda t: (0, 0)),
            pl.BlockSpec((h_dim, 2 * out_dim), lambda t: (0, 0)),
        ],
        out_specs=[
            pl.BlockSpec(
                (MB, out_dim),
                lambda t: (jnp.clip(t - NCAST - NMM, 0, NMM - 1), 0),
            ),
            pl.BlockSpec(
                (MB, out_dim),
                lambda t: (jnp.clip(t - NCAST - NMM, 0, NMM - 1), 0),
            ),
        ],
        out_shape=[
            jax.ShapeDtypeStruct((N, out_dim), jnp.float32),
            jax.ShapeDtypeStruct((N, out_dim), jnp.float32),
        ],
        scratch_shapes=[
            pltpu.VMEM((N, N), jnp.bfloat16),
            pltpu.VMEM((N, 1), jnp.float32),
            pltpu.VMEM((N, h_dim), jnp.bfloat16),
            pltpu.VMEM((N, 2 * out_dim), jnp.bfloat16),
            pltpu.VMEM((N, 256), jnp.float32),
        ],
        compiler_params=pltpu.CompilerParams(
            dimension_semantics=("arbitrary",)
        ),
    )(adj, x, W1, wcat)

    return (z_mean, z_log)
